# initial kernel scaffold (unmeasured)
import jax
import jax.numpy as jnp
from jax import lax
from jax.experimental import pallas as pl
from jax.experimental.pallas import tpu as pltpu

N_DEV = 4
SCALE = 0.08838834764831843
BLK = 64


def kernel(x, Wq, K_ext, V_ext, Wo):
    _, S, D = x.shape
    H, Dh = K_ext.shape[2], K_ext.shape[3]
    QB = 1024
    NQB = S // QB

    q = (x[0] @ Wq) * SCALE
    q = q.reshape(S, H, Dh).transpose(1, 0, 2).astype(jnp.bfloat16)
    k = K_ext[0].transpose(1, 0, 2).astype(jnp.bfloat16)
    v = V_ext[0].transpose(1, 0, 2).astype(jnp.bfloat16)
    wo = Wo.astype(jnp.bfloat16)

    def body(q_ref, k_ref, v_ref, wo_ref, out_ref,
             kg, vg, acc, kt, vt, copy_sems, tile_sems,
             ksend, krecv, vsend, vrecv):
        my = lax.axis_index("i")
        right = lax.rem(my + 1, N_DEV)
        left = lax.rem(my + N_DEV - 1, N_DEV)

        ck = pltpu.make_async_copy(k_ref, kg.at[my], copy_sems.at[0])
        cv = pltpu.make_async_copy(v_ref, vg.at[my], copy_sems.at[1])
        ck.start()
        cv.start()

        bsem = pltpu.get_barrier_semaphore()
        for nbr in (left, right):
            pl.semaphore_signal(bsem, inc=1, device_id=(nbr,),
                                device_id_type=pl.DeviceIdType.MESH)
        pl.semaphore_wait(bsem, 2)

        ck.wait()
        cv.wait()

        for h in range(N_DEV - 1):
            c = lax.rem(my + N_DEV - h, N_DEV)
            rk = pltpu.make_async_remote_copy(
                src_ref=kg.at[c], dst_ref=kg.at[c],
                send_sem=ksend.at[h], recv_sem=krecv.at[h],
                device_id=(right,), device_id_type=pl.DeviceIdType.MESH)
            rv = pltpu.make_async_remote_copy(
                src_ref=vg.at[c], dst_ref=vg.at[c],
                send_sem=vsend.at[h], recv_sem=vrecv.at[h],
                device_id=(right,), device_id_type=pl.DeviceIdType.MESH)
            rk.start()
            rv.start()
            rk.wait()
            rv.wait()

        acc[...] = jnp.zeros_like(acc)

        def qb_body(qb, _):
            row0 = my * S + qb * QB
            rb = (row0 + lax.broadcasted_iota(jnp.int32, (QB, S), 0)) // BLK

            def c_body(c, carry):
                m, l = carry
                cb = (c * S + lax.broadcasted_iota(jnp.int32, (QB, S), 1)) // BLK
                keep = (rb == cb) | (cb == 0) | (lax.rem(rb + cb, 3) == 0)
                bias = jnp.where(keep, 0.0, -1e9).astype(jnp.float32)

                def h_body(hh, carry2):
                    m, l = carry2
                    fk = pltpu.make_async_copy(kg.at[c, hh], kt, tile_sems.at[0])
                    fv = pltpu.make_async_copy(vg.at[c, hh], vt, tile_sems.at[1])
                    fk.start()
                    fv.start()
                    fk.wait()
                    fv.wait()
                    qh = q_ref[hh, pl.ds(qb * QB, QB), :]
                    s = lax.dot_general(
                        qh, kt[...], (((1,), (1,)), ((), ())),
                        preferred_element_type=jnp.float32) + bias
                    mh = m[hh]
                    cmax = jnp.max(s, axis=1, keepdims=True)
                    mn = jnp.maximum(mh, cmax)
                    alpha = jnp.exp(mh - mn)
                    p = jnp.exp(s - mn)
                    lh = alpha * l[hh] + jnp.sum(p, axis=1, keepdims=True)
                    pv = lax.dot_general(
                        p.astype(jnp.bfloat16), vt[...], (((1,), (0,)), ((), ())),
                        preferred_element_type=jnp.float32)
                    prev = acc[hh, pl.ds(qb * QB, QB), :]
                    acc[hh, pl.ds(qb * QB, QB), :] = alpha * prev + pv
                    return m.at[hh].set(mn), l.at[hh].set(lh)

                return lax.fori_loop(0, H, h_body, (m, l))

            m0 = jnp.full((H, QB, 1), -1e30, jnp.float32)
            l0 = jnp.zeros((H, QB, 1), jnp.float32)
            m, l = lax.fori_loop(0, N_DEV, c_body, (m0, l0))
            for hh in range(H):
                out_ref[pl.ds(qb * QB, QB), hh * Dh:(hh + 1) * Dh] = (
                    acc[hh, pl.ds(qb * QB, QB), :] / l[hh])
            return 0

        lax.fori_loop(0, NQB, qb_body, 0)

        ctx = out_ref[...].astype(jnp.bfloat16)
        out_ref[...] = lax.dot_general(
            ctx, wo_ref[...], (((1,), (0,)), ((), ())),
            preferred_element_type=jnp.float32)

    out = pl.pallas_call(
        body,
        out_shape=jax.ShapeDtypeStruct((S, D), jnp.float32),
        in_specs=[
            pl.BlockSpec(memory_space=pltpu.VMEM),
            pl.BlockSpec(memory_space=pltpu.HBM),
            pl.BlockSpec(memory_space=pltpu.HBM),
            pl.BlockSpec(memory_space=pltpu.VMEM),
        ],
        out_specs=pl.BlockSpec(memory_space=pltpu.VMEM),
        scratch_shapes=[
            pltpu.HBM((N_DEV, H, S, Dh), jnp.bfloat16),
            pltpu.HBM((N_DEV, H, S, Dh), jnp.bfloat16),
            pltpu.VMEM((H, S, Dh), jnp.float32),
            pltpu.VMEM((S, Dh), jnp.bfloat16),
            pltpu.VMEM((S, Dh), jnp.bfloat16),
            pltpu.SemaphoreType.DMA((2,)),
            pltpu.SemaphoreType.DMA((2,)),
            pltpu.SemaphoreType.DMA((N_DEV - 1,)),
            pltpu.SemaphoreType.DMA((N_DEV - 1,)),
            pltpu.SemaphoreType.DMA((N_DEV - 1,)),
            pltpu.SemaphoreType.DMA((N_DEV - 1,)),
        ],
        compiler_params=pltpu.CompilerParams(collective_id=0),
    )(q, k, v, wo)
    return out[None]


# baseline (device time: 1050815 ns/iter reference)
import jax
import jax.numpy as jnp
from jax import lax
from jax.experimental import pallas as pl
from jax.experimental.pallas import tpu as pltpu

N_DEV = 4
SCALE = 0.08838834764831843
BLK = 64


def kernel(x, Wq, K_ext, V_ext, Wo):
    _, S, D = x.shape
    H, Dh = K_ext.shape[2], K_ext.shape[3]
    QB = 512
    NQB = S // QB

    q = (x[0] @ Wq) * SCALE
    q = q.reshape(S, H, Dh).transpose(1, 0, 2).astype(jnp.bfloat16)
    k = K_ext[0].transpose(1, 0, 2).astype(jnp.bfloat16)
    v = V_ext[0].transpose(1, 0, 2).astype(jnp.bfloat16)
    wo = Wo.astype(jnp.bfloat16)

    def body(q_ref, k_ref, v_ref, wo_ref, out_ref, kg, vg,
             acc, kt, vt, copy_sems, tile_sems,
             ksend, krecv, vsend, vrecv):
        my = lax.axis_index("i")
        right = lax.rem(my + 1, N_DEV)
        left = lax.rem(my + N_DEV - 1, N_DEV)

        ck = pltpu.make_async_copy(k_ref, kg.at[my], copy_sems.at[0])
        cv = pltpu.make_async_copy(v_ref, vg.at[my], copy_sems.at[1])
        ck.start()
        cv.start()

        bsem = pltpu.get_barrier_semaphore()
        for nbr in (left, right):
            pl.semaphore_signal(bsem, inc=1, device_id=(nbr,),
                                device_id_type=pl.DeviceIdType.MESH)
        pl.semaphore_wait(bsem, 2)

        ck.wait()
        cv.wait()

        for h in range(N_DEV - 1):
            c = lax.rem(my + N_DEV - h, N_DEV)
            rk = pltpu.make_async_remote_copy(
                src_ref=kg.at[c], dst_ref=kg.at[c],
                send_sem=ksend.at[h], recv_sem=krecv.at[h],
                device_id=(right,), device_id_type=pl.DeviceIdType.MESH)
            rv = pltpu.make_async_remote_copy(
                src_ref=vg.at[c], dst_ref=vg.at[c],
                send_sem=vsend.at[h], recv_sem=vrecv.at[h],
                device_id=(right,), device_id_type=pl.DeviceIdType.MESH)
            rk.start()
            rv.start()
            rk.wait()
            rv.wait()

        acc[...] = jnp.zeros_like(acc)

        def qb_body(qb, _):
            row0 = my * S + qb * QB
            rb = (row0 + lax.broadcasted_iota(jnp.int32, (QB, S), 0)) // BLK

            def c_body(c, carry):
                ms, ls = carry
                cb = (c * S + lax.broadcasted_iota(jnp.int32, (QB, S), 1)) // BLK
                keep = (rb == cb) | (cb == 0) | (lax.rem(rb + cb, 3) == 0)
                bias = jnp.where(keep, 0.0, -1e9).astype(jnp.float32)

                new_ms, new_ls = [], []
                for hh in range(H):
                    fk = pltpu.make_async_copy(kg.at[c, hh], kt, tile_sems.at[0])
                    fv = pltpu.make_async_copy(vg.at[c, hh], vt, tile_sems.at[1])
                    fk.start()
                    fv.start()
                    fk.wait()
                    fv.wait()
                    qh = q_ref[hh, pl.ds(qb * QB, QB), :]
                    s = lax.dot_general(
                        qh, kt[...], (((1,), (1,)), ((), ())),
                        preferred_element_type=jnp.float32) + bias
                    mh = ms[hh]
                    cmax = jnp.max(s, axis=1, keepdims=True)
                    mn = jnp.maximum(mh, cmax)
                    alpha = jnp.exp(mh - mn)
                    p = jnp.exp(s - mn)
                    lh = alpha * ls[hh] + jnp.sum(p, axis=1, keepdims=True)
                    pv = lax.dot_general(
                        p.astype(jnp.bfloat16), vt[...], (((1,), (0,)), ((), ())),
                        preferred_element_type=jnp.float32)
                    prev = acc[hh, pl.ds(qb * QB, QB), :]
                    acc[hh, pl.ds(qb * QB, QB), :] = alpha * prev + pv
                    new_ms.append(mn)
                    new_ls.append(lh)
                return tuple(new_ms), tuple(new_ls)

            m0 = tuple(jnp.full((QB, 1), -1e30, jnp.float32) for _ in range(H))
            l0 = tuple(jnp.zeros((QB, 1), jnp.float32) for _ in range(H))
            ms, ls = lax.fori_loop(0, N_DEV, c_body, (m0, l0))
            for hh in range(H):
                out_ref[pl.ds(qb * QB, QB), hh * Dh:(hh + 1) * Dh] = (
                    acc[hh, pl.ds(qb * QB, QB), :] / ls[hh])
            return 0

        lax.fori_loop(0, NQB, qb_body, 0)

        ctx = out_ref[...].astype(jnp.bfloat16)
        out_ref[...] = lax.dot_general(
            ctx, wo_ref[...], (((1,), (0,)), ((), ())),
            preferred_element_type=jnp.float32)

    out, _, _ = pl.pallas_call(
        body,
        out_shape=(
            jax.ShapeDtypeStruct((S, D), jnp.float32),
            jax.ShapeDtypeStruct((N_DEV, H, S, Dh), jnp.bfloat16),
            jax.ShapeDtypeStruct((N_DEV, H, S, Dh), jnp.bfloat16),
        ),
        in_specs=[
            pl.BlockSpec(memory_space=pltpu.VMEM),
            pl.BlockSpec(memory_space=pltpu.HBM),
            pl.BlockSpec(memory_space=pltpu.HBM),
            pl.BlockSpec(memory_space=pltpu.VMEM),
        ],
        out_specs=(
            pl.BlockSpec(memory_space=pltpu.VMEM),
            pl.BlockSpec(memory_space=pltpu.HBM),
            pl.BlockSpec(memory_space=pltpu.HBM),
        ),
        scratch_shapes=[
            pltpu.VMEM((H, S, Dh), jnp.float32),
            pltpu.VMEM((S, Dh), jnp.bfloat16),
            pltpu.VMEM((S, Dh), jnp.bfloat16),
            pltpu.SemaphoreType.DMA((2,)),
            pltpu.SemaphoreType.DMA((2,)),
            pltpu.SemaphoreType.DMA((N_DEV - 1,)),
            pltpu.SemaphoreType.DMA((N_DEV - 1,)),
            pltpu.SemaphoreType.DMA((N_DEV - 1,)),
            pltpu.SemaphoreType.DMA((N_DEV - 1,)),
        ],
        compiler_params=pltpu.CompilerParams(
            collective_id=0, vmem_limit_bytes=56 * 1024 * 1024),
    )(q, k, v, wo)
    return out[None]


# device time: 713230 ns/iter; 1.4733x vs baseline; 1.4733x over previous
import jax
import jax.numpy as jnp
from jax import lax
from jax.experimental import pallas as pl
from jax.experimental.pallas import tpu as pltpu

N_DEV = 4
SCALE = 0.08838834764831843
BLK = 64


def kernel(x, Wq, K_ext, V_ext, Wo):
    _, S, D = x.shape
    H, Dh = K_ext.shape[2], K_ext.shape[3]
    QB = 512
    NQB = S // QB

    q = (x[0] @ Wq) * SCALE
    q = q.reshape(S, H, Dh).transpose(1, 0, 2).astype(jnp.bfloat16)
    k = K_ext[0].transpose(1, 0, 2).astype(jnp.bfloat16)
    v = V_ext[0].transpose(1, 0, 2).astype(jnp.bfloat16)
    wo = Wo.astype(jnp.bfloat16)

    def body(q_ref, k_ref, v_ref, wo_ref, out_ref, kg, vg,
             acc, kc, vc, m_sc, l_sc, copy_sems, chunk_sems,
             ksend, krecv, vsend, vrecv):
        my = lax.axis_index("i")

        ck = pltpu.make_async_copy(k_ref, kg.at[my], copy_sems.at[0])
        cv = pltpu.make_async_copy(v_ref, vg.at[my], copy_sems.at[1])
        ck.start()
        cv.start()

        bsem = pltpu.get_barrier_semaphore()
        for off in (1, 2, 3):
            pl.semaphore_signal(bsem, inc=1,
                                device_id=(lax.rem(my + off, N_DEV),),
                                device_id_type=pl.DeviceIdType.MESH)
        pl.semaphore_wait(bsem, 3)

        ck.wait()
        cv.wait()

        sends = []
        for off in (1, 2, 3):
            p = lax.rem(my + off, N_DEV)
            slot = 3 - off
            rk = pltpu.make_async_remote_copy(
                src_ref=kg.at[my], dst_ref=kg.at[my],
                send_sem=ksend.at[slot], recv_sem=krecv.at[slot],
                device_id=(p,), device_id_type=pl.DeviceIdType.MESH)
            rv = pltpu.make_async_remote_copy(
                src_ref=vg.at[my], dst_ref=vg.at[my],
                send_sem=vsend.at[slot], recv_sem=vrecv.at[slot],
                device_id=(p,), device_id_type=pl.DeviceIdType.MESH)
            rk.start()
            rv.start()
            sends += [rk, rv]

        acc[...] = jnp.zeros_like(acc)
        m_sc[...] = jnp.full_like(m_sc, -1e30)
        l_sc[...] = jnp.zeros_like(l_sc)

        for off in (0, 1, 3, 2):
            c = lax.rem(my + off, N_DEV)
            if off != 0:
                slot = off - 1
                wk = pltpu.make_async_remote_copy(
                    src_ref=kg.at[c], dst_ref=kg.at[c],
                    send_sem=ksend.at[slot], recv_sem=krecv.at[slot],
                    device_id=(my,), device_id_type=pl.DeviceIdType.MESH)
                wv = pltpu.make_async_remote_copy(
                    src_ref=vg.at[c], dst_ref=vg.at[c],
                    send_sem=vsend.at[slot], recv_sem=vrecv.at[slot],
                    device_id=(my,), device_id_type=pl.DeviceIdType.MESH)
                wk.wait_recv()
                wv.wait_recv()

            fk = pltpu.make_async_copy(kg.at[c], kc, chunk_sems.at[0])
            fv = pltpu.make_async_copy(vg.at[c], vc, chunk_sems.at[1])
            fk.start()
            fv.start()
            fk.wait()
            fv.wait()

            def qb_body(qb, _, c=c):
                row0 = my * S + qb * QB
                rb = (row0 + lax.broadcasted_iota(jnp.int32, (QB, S), 0)) // BLK
                cb = (c * S + lax.broadcasted_iota(jnp.int32, (QB, S), 1)) // BLK
                keep = (rb == cb) | (cb == 0) | (lax.rem(rb + cb, 3) == 0)
                bias = jnp.where(keep, 0.0, -1e9).astype(jnp.float32)

                for hh in range(H):
                    qh = q_ref[hh, pl.ds(qb * QB, QB), :]
                    s = lax.dot_general(
                        qh, kc[hh], (((1,), (1,)), ((), ())),
                        preferred_element_type=jnp.float32) + bias
                    mh = jnp.reshape(m_sc[hh, qb, :], (QB, 1))
                    lh = jnp.reshape(l_sc[hh, qb, :], (QB, 1))
                    cmax = jnp.max(s, axis=1, keepdims=True)
                    mn = jnp.maximum(mh, cmax)
                    alpha = jnp.exp(mh - mn)
                    p = jnp.exp(s - mn)
                    ln = alpha * lh + jnp.sum(p, axis=1, keepdims=True)
                    pv = lax.dot_general(
                        p.astype(jnp.bfloat16), vc[hh], (((1,), (0,)), ((), ())),
                        preferred_element_type=jnp.float32)
                    prev = acc[hh, pl.ds(qb * QB, QB), :]
                    acc[hh, pl.ds(qb * QB, QB), :] = alpha * prev + pv
                    m_sc[hh, qb, :] = jnp.reshape(mn, (QB,))
                    l_sc[hh, qb, :] = jnp.reshape(ln, (QB,))
                return 0

            lax.fori_loop(0, NQB, qb_body, 0)

        for r in sends:
            r.wait_send()

        def norm_body(qb, _):
            for hh in range(H):
                lh = jnp.reshape(l_sc[hh, qb, :], (QB, 1))
                out_ref[pl.ds(qb * QB, QB), hh * Dh:(hh + 1) * Dh] = (
                    acc[hh, pl.ds(qb * QB, QB), :] / lh)
            return 0

        lax.fori_loop(0, NQB, norm_body, 0)

        ctx = out_ref[...].astype(jnp.bfloat16)
        out_ref[...] = lax.dot_general(
            ctx, wo_ref[...], (((1,), (0,)), ((), ())),
            preferred_element_type=jnp.float32)

    out, _, _ = pl.pallas_call(
        body,
        out_shape=(
            jax.ShapeDtypeStruct((S, D), jnp.float32),
            jax.ShapeDtypeStruct((N_DEV, H, S, Dh), jnp.bfloat16),
            jax.ShapeDtypeStruct((N_DEV, H, S, Dh), jnp.bfloat16),
        ),
        in_specs=[
            pl.BlockSpec(memory_space=pltpu.VMEM),
            pl.BlockSpec(memory_space=pltpu.HBM),
            pl.BlockSpec(memory_space=pltpu.HBM),
            pl.BlockSpec(memory_space=pltpu.VMEM),
        ],
        out_specs=(
            pl.BlockSpec(memory_space=pltpu.VMEM),
            pl.BlockSpec(memory_space=pltpu.HBM),
            pl.BlockSpec(memory_space=pltpu.HBM),
        ),
        scratch_shapes=[
            pltpu.VMEM((H, S, Dh), jnp.float32),
            pltpu.VMEM((H, S, Dh), jnp.bfloat16),
            pltpu.VMEM((H, S, Dh), jnp.bfloat16),
            pltpu.VMEM((H, NQB, QB), jnp.float32),
            pltpu.VMEM((H, NQB, QB), jnp.float32),
            pltpu.SemaphoreType.DMA((2,)),
            pltpu.SemaphoreType.DMA((2,)),
            pltpu.SemaphoreType.DMA((3,)),
            pltpu.SemaphoreType.DMA((3,)),
            pltpu.SemaphoreType.DMA((3,)),
            pltpu.SemaphoreType.DMA((3,)),
        ],
        compiler_params=pltpu.CompilerParams(
            collective_id=0, vmem_limit_bytes=56 * 1024 * 1024),
    )(q, k, v, wo)
    return out[None]
